# baseline (speedup 1.0000x reference)
import functools

import jax
import jax.numpy as jnp
from jax import lax
from jax.experimental import pallas as pl
from jax.experimental.pallas import tpu as pltpu
from jax.experimental.pallas import tpu_sc as plsc

N = 100000
D = 128
NW = 32
CHUNK = 256
NCH = N // CHUNK
FULL_T = NCH // NW
REM = NCH - FULL_T * NW
TAIL = N - NCH * CHUNK
TAIL_BASE = NCH * CHUNK
TAIL_WID = NW - 1

_mesh = plsc.VectorSubcoreMesh(core_axis_name="c", subcore_axis_name="s")


@functools.partial(
    pl.kernel,
    mesh=_mesh,
    out_type=jax.ShapeDtypeStruct((N, D), jnp.float32),
    scratch_types=[
        pltpu.VMEM((CHUNK,), jnp.int32),
        pltpu.VMEM((CHUNK,), jnp.int32),
        pltpu.VMEM((CHUNK, D), jnp.float32),
        pltpu.VMEM((CHUNK, D), jnp.float32),
        pltpu.VMEM((TAIL,), jnp.int32),
        pltpu.VMEM((TAIL, D), jnp.float32),
        pltpu.VMEM((10, D), jnp.float32),
        pltpu.SemaphoreType.DMA,
    ],
)
def _embed_gather(idx_hbm, table_hbm, out_hbm, idx0, idx1, rows0, rows1,
                  idx_t, rows_t, table_v, sem_s):
    wid = lax.axis_index("s") * 2 + lax.axis_index("c")
    idx_bufs = (idx0, idx1)
    rows_bufs = (rows0, rows1)

    pltpu.sync_copy(table_hbm, table_v)

    def base(t):
        return (wid + t * NW) * CHUNK

    def load_idx(t, b):
        pltpu.sync_copy(idx_hbm.at[pl.ds(base(t), CHUNK)], idx_bufs[b])

    def build_rows(idx_v, rows_v, nrows):
        def group_body(g, carry):
            ivec = idx_v[pl.ds(g * 16, 16)]
            for l in range(16):
                ir = ivec[l]
                r = g * 16 + l
                for j in range(D // 16):
                    rows_v[r, pl.ds(j * 16, 16)] = \
                        table_v[ir, pl.ds(j * 16, 16)]
            return carry
        lax.fori_loop(0, nrows // 16, group_body, 0)

    chunk_wait = pltpu.make_async_copy(
        rows0, out_hbm.at[pl.ds(0, CHUNK)], sem_s)

    def do_round(t, b, first):
        if not first:
            chunk_wait.wait()
        build_rows(idx_bufs[b], rows_bufs[b], CHUNK)
        pltpu.async_copy(rows_bufs[b], out_hbm.at[pl.ds(base(t), CHUNK)],
                         sem_s)

    load_idx(0, 0)
    load_idx(1, 1)

    def pair_body(k, carry):
        t = 2 * k

        @pl.when(k > 0)
        def _():
            chunk_wait.wait()
        do_round(t, 0, first=True)

        @pl.when(t + 2 < FULL_T)
        def _():
            load_idx(t + 2, 0)

        @pl.when(k > 0)
        def _():
            chunk_wait.wait()
        do_round(t + 1, 1, first=True)

        @pl.when(t + 3 < FULL_T)
        def _():
            load_idx(t + 3, 1)
        return carry

    lax.fori_loop(0, FULL_T // 2, pair_body, 0)

    @pl.when(wid < REM)
    def _():
        load_idx(FULL_T, 0)
        do_round(FULL_T, 0, first=False)

    @pl.when(wid == TAIL_WID)
    def _():
        pltpu.sync_copy(idx_hbm.at[pl.ds(TAIL_BASE, TAIL)], idx_t)
        build_rows(idx_t, rows_t, TAIL)
        pltpu.async_copy(rows_t, out_hbm.at[pl.ds(TAIL_BASE, TAIL)],
                         sem_s).wait()

    chunk_wait.wait()
    chunk_wait.wait()


def kernel(centrality, embedding):
    idx = centrality.astype(jnp.int32)
    return _embed_gather(idx, embedding)


# speedup vs baseline: 2.9868x; 2.9868x over previous
import functools

import jax
import jax.numpy as jnp
from jax import lax
from jax.experimental import pallas as pl
from jax.experimental.pallas import tpu as pltpu
from jax.experimental.pallas import tpu_sc as plsc

N = 100000
D = 128
NW = 32
CHUNK = 128
NCH = N // CHUNK
FULL_T = NCH // NW
REM = NCH - FULL_T * NW
TAIL = N - NCH * CHUNK
TAIL_BASE = NCH * CHUNK
TAIL_WID = NW - 1
NBUF = 5

_mesh = plsc.VectorSubcoreMesh(core_axis_name="c", subcore_axis_name="s")


@functools.partial(
    pl.kernel,
    mesh=_mesh,
    out_type=jax.ShapeDtypeStruct((N, D), jnp.float32),
    scratch_types=(
        [pltpu.VMEM((CHUNK,), jnp.int32) for _ in range(NBUF)]
        + [pltpu.VMEM((CHUNK, D), jnp.float32) for _ in range(NBUF)]
        + [pltpu.VMEM((TAIL,), jnp.int32),
           pltpu.VMEM((TAIL, D), jnp.float32),
           pltpu.VMEM_SHARED((10, D), jnp.float32),
           pltpu.SemaphoreType.DMA,
           pltpu.SemaphoreType.DMA]
    ),
)
def _embed_gather(idx_hbm, table_hbm, out_hbm,
                  ib0, ib1, ib2, ib3, ib4, rb0, rb1, rb2, rb3, rb4,
                  idx_t, rows_t, table_sh, sem_g, sem_s):
    wid = lax.axis_index("s") * 2 + lax.axis_index("c")
    idx_bufs = (ib0, ib1, ib2, ib3, ib4)
    rows_bufs = (rb0, rb1, rb2, rb3, rb4)

    @pl.when(lax.axis_index("s") == 0)
    def _():
        pltpu.sync_copy(table_hbm, table_sh)

    def base(t):
        return (wid + t * NW) * CHUNK

    def load_idx(t):
        pltpu.sync_copy(idx_hbm.at[pl.ds(base(t), CHUNK)], idx_bufs[t % NBUF])

    def start_gather(t):
        return pltpu.async_copy(table_sh.at[idx_bufs[t % NBUF]],
                                rows_bufs[t % NBUF], sem_g)

    g_h = [None] * (FULL_T + 1)
    scat_h = [None] * (FULL_T + 1)

    load_idx(0)
    load_idx(1)
    load_idx(2)
    plsc.subcore_barrier()
    g_h[0] = start_gather(0)
    g_h[1] = start_gather(1)

    for t in range(FULL_T):
        if t >= 3:
            scat_h[t - 3].wait()
        if t + 2 < FULL_T:
            g_h[t + 2] = start_gather(t + 2)
        elif t + 2 == FULL_T:
            @pl.when(wid < REM)
            def _():
                g_h[FULL_T] = start_gather(FULL_T)
        g_h[t].wait()
        scat_h[t] = pltpu.async_copy(
            rows_bufs[t % NBUF], out_hbm.at[pl.ds(base(t), CHUNK)], sem_s)
        if t + 3 < FULL_T:
            load_idx(t + 3)
        elif t + 3 == FULL_T:
            @pl.when(wid < REM)
            def _():
                load_idx(FULL_T)

    @pl.when(wid < REM)
    def _():
        g_h[FULL_T].wait()
        pltpu.async_copy(rows_bufs[FULL_T % NBUF],
                         out_hbm.at[pl.ds(base(FULL_T), CHUNK)], sem_s)
        scat_h[FULL_T - 3].wait()

    @pl.when(wid == TAIL_WID)
    def _():
        pltpu.sync_copy(idx_hbm.at[pl.ds(TAIL_BASE, TAIL)], idx_t)
        pltpu.async_copy(table_sh.at[idx_t], rows_t, sem_g).wait()
        pltpu.async_copy(rows_t, out_hbm.at[pl.ds(TAIL_BASE, TAIL)],
                         sem_s).wait()

    scat_h[FULL_T - 3].wait()
    scat_h[FULL_T - 2].wait()
    scat_h[FULL_T - 1].wait()


def kernel(centrality, embedding):
    idx = centrality.astype(jnp.int32)
    return _embed_gather(idx, embedding)


# speedup vs baseline: 3.0375x; 1.0169x over previous
import functools

import jax
import jax.numpy as jnp
from jax import lax
from jax.experimental import pallas as pl
from jax.experimental.pallas import tpu as pltpu
from jax.experimental.pallas import tpu_sc as plsc

N = 100000
D = 128
NW = 32
CHUNK = 256
NCH = N // CHUNK
FULL_T = NCH // NW
REM = NCH - FULL_T * NW
TAIL = N - NCH * CHUNK
TAIL_BASE = NCH * CHUNK
TAIL_WID = NW - 1
NBUF = 3

_mesh = plsc.VectorSubcoreMesh(core_axis_name="c", subcore_axis_name="s")


@functools.partial(
    pl.kernel,
    mesh=_mesh,
    out_type=jax.ShapeDtypeStruct((N, D), jnp.float32),
    scratch_types=[
        pltpu.VMEM((CHUNK,), jnp.int32),
        pltpu.VMEM((CHUNK,), jnp.int32),
        pltpu.VMEM((CHUNK,), jnp.int32),
        pltpu.VMEM((CHUNK, D), jnp.float32),
        pltpu.VMEM((CHUNK, D), jnp.float32),
        pltpu.VMEM((CHUNK, D), jnp.float32),
        pltpu.VMEM((TAIL,), jnp.int32),
        pltpu.VMEM((TAIL, D), jnp.float32),
        pltpu.VMEM_SHARED((10, D), jnp.float32),
        pltpu.SemaphoreType.DMA,
        pltpu.SemaphoreType.DMA,
    ],
)
def _embed_gather(idx_hbm, table_hbm, out_hbm, idx0, idx1, idx2,
                  rows0, rows1, rows2, idx_t, rows_t, table_sh,
                  sem_g, sem_s):
    wid = lax.axis_index("s") * 2 + lax.axis_index("c")
    idx_bufs = (idx0, idx1, idx2)
    rows_bufs = (rows0, rows1, rows2)

    @pl.when(lax.axis_index("s") == 0)
    def _():
        pltpu.sync_copy(table_hbm, table_sh)

    def base(t):
        return (wid + t * NW) * CHUNK

    def load_idx(t):
        pltpu.sync_copy(idx_hbm.at[pl.ds(base(t), CHUNK)], idx_bufs[t % NBUF])

    def start_gather(t):
        return pltpu.async_copy(table_sh.at[idx_bufs[t % NBUF]],
                                rows_bufs[t % NBUF], sem_g)

    g_h = [None] * (FULL_T + 1)
    scat_h = [None] * (FULL_T + 1)

    load_idx(0)
    load_idx(1)
    plsc.subcore_barrier()
    g_h[0] = start_gather(0)

    for t in range(FULL_T):
        if t >= 2:
            scat_h[t - 2].wait()
        if t + 1 < FULL_T:
            g_h[t + 1] = start_gather(t + 1)
        elif t + 1 == FULL_T:
            @pl.when(wid < REM)
            def _():
                g_h[FULL_T] = start_gather(FULL_T)
        g_h[t].wait()
        scat_h[t] = pltpu.async_copy(
            rows_bufs[t % NBUF], out_hbm.at[pl.ds(base(t), CHUNK)], sem_s)
        if t + 2 < FULL_T:
            load_idx(t + 2)
        elif t + 2 == FULL_T:
            @pl.when(wid < REM)
            def _():
                load_idx(FULL_T)

    @pl.when(wid < REM)
    def _():
        g_h[FULL_T].wait()
        pltpu.async_copy(rows_bufs[FULL_T % NBUF],
                         out_hbm.at[pl.ds(base(FULL_T), CHUNK)], sem_s)
        scat_h[FULL_T - 2].wait()

    @pl.when(wid == TAIL_WID)
    def _():
        pltpu.sync_copy(idx_hbm.at[pl.ds(TAIL_BASE, TAIL)], idx_t)
        pltpu.async_copy(table_sh.at[idx_t], rows_t, sem_g).wait()
        pltpu.async_copy(rows_t, out_hbm.at[pl.ds(TAIL_BASE, TAIL)],
                         sem_s).wait()

    scat_h[FULL_T - 2].wait()
    scat_h[FULL_T - 1].wait()


def kernel(centrality, embedding):
    idx = centrality.astype(jnp.int32)
    return _embed_gather(idx, embedding)
